# baseline (speedup 1.0000x reference)
import functools

import jax
import jax.numpy as jnp
import numpy as np
from jax import lax
from jax.experimental import pallas as pl
from jax.experimental.pallas import tpu as pltpu
from jax.experimental.pallas import tpu_sc as plsc

_NUM_EMB = 32128
_DIM = 512
_BATCH = 4
_TOKENS = 8192
_FLAT = _BATCH * _TOKENS
_NW = 32
_POS_W = _TOKENS // _NW
_CHUNK = 32
_NCHUNK = _POS_W // _CHUNK
_NSTEP = _NCHUNK * _BATCH


def _pos_encoding(token_length, embedding_dim):
    pos = np.arange(token_length, dtype=np.float32)[:, None]
    i = np.arange(embedding_dim, dtype=np.float32)[None, :]
    angle_rates = 1.0 / np.power(
        10000.0, (2.0 * np.floor(i / 2.0)) / float(embedding_dim))
    angles = pos * angle_rates
    pe = np.where((np.arange(embedding_dim)[None, :] % 2) == 0,
                  np.sin(angles), np.cos(angles))
    return pe.astype(np.float32)


_PE = _pos_encoding(_TOKENS, _DIM)


_mesh = plsc.VectorSubcoreMesh(core_axis_name="c", subcore_axis_name="s")


@functools.partial(
    pl.kernel,
    out_type=jax.ShapeDtypeStruct((_FLAT, _DIM), jnp.float32),
    mesh=_mesh,
    compiler_params=pltpu.CompilerParams(use_tc_tiling_on_sc=True),
    scratch_types=[
        pltpu.VMEM((_BATCH, _POS_W), jnp.int32),
        pltpu.VMEM((_CHUNK, _DIM), jnp.float32),
        pltpu.VMEM((_CHUNK, _DIM), jnp.float32),
        pltpu.VMEM((_CHUNK, _DIM), jnp.float32),
        pltpu.VMEM((_CHUNK, _DIM), jnp.float32),
        pltpu.VMEM((_CHUNK, _DIM), jnp.float32),
        pltpu.VMEM((_CHUNK, _DIM), jnp.float32),
        pltpu.SemaphoreType.DMA,
        pltpu.SemaphoreType.DMA,
        pltpu.SemaphoreType.DMA,
        pltpu.SemaphoreType.DMA,
        pltpu.SemaphoreType.DMA,
        pltpu.SemaphoreType.DMA,
        pltpu.SemaphoreType.DMA,
        pltpu.SemaphoreType.DMA,
        pltpu.SemaphoreType.DMA,
        pltpu.SemaphoreType.DMA,
    ],
)
def _embed_sc(table_hbm, idx_hbm, pe_hbm, out_hbm, idx_v,
              ob0, ob1, ob2, ob3, pb0, pb1,
              gs0, gs1, gs2, gs3, ps0, ps1, os0, os1, os2, os3):
    obuf = (ob0, ob1, ob2, ob3)
    pbuf = (pb0, pb1)
    gsem = (gs0, gs1, gs2, gs3)
    psem = (ps0, ps1)
    osem = (os0, os1, os2, os3)

    wid = lax.axis_index("s") * _mesh.num_cores + lax.axis_index("c")
    pos_base = wid * _POS_W
    for b in range(_BATCH):
        pltpu.sync_copy(idx_hbm.at[b, wid], idx_v.at[b])

    def out_slice(c, b):
        return out_hbm.at[
            pl.ds(b * _TOKENS + pos_base + c * _CHUNK, _CHUNK)]

    def pe_slice(c):
        return pe_hbm.at[pl.ds(pos_base + c * _CHUNK, _CHUNK)]

    def idx_slice(c, b):
        return idx_v.at[b, pl.ds(c * _CHUNK, _CHUNK)]

    def start_gather(c, b, i):
        pltpu.async_copy(table_hbm.at[idx_slice(c, b)], obuf[i], gsem[i])

    def wait_gather(c, b, i):
        pltpu.make_async_copy(table_hbm.at[idx_slice(c, b)], obuf[i],
                              gsem[i]).wait()

    def start_pe(c, j):
        pltpu.async_copy(pe_slice(c), pbuf[j], psem[j])

    def wait_pe(c, j):
        pltpu.make_async_copy(pe_slice(c), pbuf[j], psem[j]).wait()

    def start_write(c, b, i):
        pltpu.async_copy(obuf[i], out_slice(c, b), osem[i])

    def wait_write(c, b, i):
        pltpu.make_async_copy(obuf[i], out_slice(c, b), osem[i]).wait()

    start_pe(0, 0)
    start_gather(0, 0, 0)
    start_gather(0, 1, 1)

    @pl.loop(0, _NSTEP, step=8)
    def step_loop(s0):
        s0d4 = s0 // 4
        for sb in range(8):
            i = sb % 4
            b = sb % 4
            j = (sb // 4) % 2
            s = s0 + sb
            c = s0d4 + (sb // 4)

            @pl.when(s >= 2)
            def _():
                wait_write(s0d4 + ((sb - 2) // 4), (sb - 2) % 4,
                           (sb - 2) % 4)

            @pl.when(s + 2 < _NSTEP)
            def _():
                start_gather(s0d4 + ((sb + 2) // 4), (sb + 2) % 4,
                             (i + 2) % 4)

            if b == 0:
                wait_pe(c, j)

                @pl.when(c + 1 < _NCHUNK)
                def _():
                    start_pe(c + 1, 1 - j)

            wait_gather(c, b, i)

            @plsc.parallel_loop(0, _CHUNK, step=1, unroll=1)
            def row_body(r):
                for k in range(_DIM // 16):
                    plsc.addupdate(obuf[i].at[r, pl.ds(k * 16, 16)],
                                   pbuf[j][r, pl.ds(k * 16, 16)])

            start_write(c, b, i)

    wait_write(_NCHUNK - 1, 2, 2)
    wait_write(_NCHUNK - 1, 3, 3)


@jax.jit
def kernel(x, table):
    pe = jnp.asarray(_PE)
    idx = x.reshape(_BATCH, _NW, _POS_W)
    out = _embed_sc(table, idx, pe)
    return out.reshape(_BATCH, _TOKENS, _DIM)


# speedup vs baseline: 1.0257x; 1.0257x over previous
import functools

import jax
import jax.numpy as jnp
import numpy as np
from jax import lax
from jax.experimental import pallas as pl
from jax.experimental.pallas import tpu as pltpu
from jax.experimental.pallas import tpu_sc as plsc

_NUM_EMB = 32128
_DIM = 512
_BATCH = 4
_TOKENS = 8192
_FLAT = _BATCH * _TOKENS
_NW = 32
_POS_W = _TOKENS // _NW
_CHUNK = 32
_NCHUNK = _POS_W // _CHUNK
_NSTEP = _NCHUNK * _BATCH


def _pos_encoding(token_length, embedding_dim):
    pos = np.arange(token_length, dtype=np.float32)[:, None]
    i = np.arange(embedding_dim, dtype=np.float32)[None, :]
    angle_rates = 1.0 / np.power(
        10000.0, (2.0 * np.floor(i / 2.0)) / float(embedding_dim))
    angles = pos * angle_rates
    pe = np.where((np.arange(embedding_dim)[None, :] % 2) == 0,
                  np.sin(angles), np.cos(angles))
    return pe.astype(np.float32)


_PE = _pos_encoding(_TOKENS, _DIM)


_mesh = plsc.VectorSubcoreMesh(core_axis_name="c", subcore_axis_name="s")


@functools.partial(
    pl.kernel,
    out_type=jax.ShapeDtypeStruct((_FLAT, _DIM), jnp.float32),
    mesh=_mesh,
    scratch_types=[
        pltpu.VMEM((_NSTEP, _CHUNK), jnp.int32),
        pltpu.VMEM((_CHUNK, _DIM), jnp.float32),
        pltpu.VMEM((_CHUNK, _DIM), jnp.float32),
        pltpu.VMEM((_CHUNK, _DIM), jnp.float32),
        pltpu.VMEM((_CHUNK, _DIM), jnp.float32),
        pltpu.VMEM((_CHUNK, _DIM), jnp.float32),
        pltpu.VMEM((_CHUNK, _DIM), jnp.float32),
        pltpu.SemaphoreType.DMA,
        pltpu.SemaphoreType.DMA,
        pltpu.SemaphoreType.DMA,
        pltpu.SemaphoreType.DMA,
        pltpu.SemaphoreType.DMA,
        pltpu.SemaphoreType.DMA,
        pltpu.SemaphoreType.DMA,
        pltpu.SemaphoreType.DMA,
        pltpu.SemaphoreType.DMA,
        pltpu.SemaphoreType.DMA,
    ],
)
def _embed_sc(table_hbm, idx_hbm, pe_hbm, out_hbm, idx_v,
              ob0, ob1, ob2, ob3, pb0, pb1,
              gs0, gs1, gs2, gs3, ps0, ps1, os0, os1, os2, os3):
    obuf = (ob0, ob1, ob2, ob3)
    pbuf = (pb0, pb1)
    gsem = (gs0, gs1, gs2, gs3)
    psem = (ps0, ps1)
    osem = (os0, os1, os2, os3)

    wid = lax.axis_index("s") * _mesh.num_cores + lax.axis_index("c")
    pos_base = wid * _POS_W
    pltpu.sync_copy(idx_hbm.at[wid], idx_v)

    def out_slice(c, b):
        return out_hbm.at[
            pl.ds(b * _TOKENS + pos_base + c * _CHUNK, _CHUNK)]

    def pe_slice(c):
        return pe_hbm.at[pl.ds(pos_base + c * _CHUNK, _CHUNK)]

    def start_gather(s, i):
        pltpu.async_copy(table_hbm.at[idx_v.at[s]], obuf[i], gsem[i])

    def wait_gather(s, i):
        pltpu.make_async_copy(table_hbm.at[idx_v.at[s]], obuf[i],
                              gsem[i]).wait()

    def start_pe(c, j):
        pltpu.async_copy(pe_slice(c), pbuf[j], psem[j])

    def wait_pe(c, j):
        pltpu.make_async_copy(pe_slice(c), pbuf[j], psem[j]).wait()

    def start_write(c, b, i):
        pltpu.async_copy(obuf[i], out_slice(c, b), osem[i])

    def wait_write(c, b, i):
        pltpu.make_async_copy(obuf[i], out_slice(c, b), osem[i]).wait()

    start_pe(0, 0)
    start_gather(0, 0)
    start_gather(1, 1)

    @pl.loop(0, _NSTEP, step=8)
    def step_loop(s0):
        s0d4 = s0 // 4
        for sb in range(8):
            i = sb % 4
            b = sb % 4
            j = (sb // 4) % 2
            s = s0 + sb
            c = s0d4 + (sb // 4)

            @pl.when(s >= 2)
            def _():
                wait_write(s0d4 + ((sb - 2) // 4), (sb - 2) % 4,
                           (sb - 2) % 4)

            @pl.when(s + 2 < _NSTEP)
            def _():
                start_gather(s + 2, (i + 2) % 4)

            if b == 0:
                wait_pe(c, j)

                @pl.when(c + 1 < _NCHUNK)
                def _():
                    start_pe(c + 1, 1 - j)

            wait_gather(s, i)

            @plsc.parallel_loop(0, _CHUNK, step=1, unroll=1)
            def row_body(r):
                for k in range(_DIM // 16):
                    plsc.addupdate(obuf[i].at[r, pl.ds(k * 16, 16)],
                                   pbuf[j][r, pl.ds(k * 16, 16)])

            start_write(c, b, i)

    wait_write(_NCHUNK - 1, 2, 2)
    wait_write(_NCHUNK - 1, 3, 3)


@jax.jit
def kernel(x, table):
    pe = jnp.asarray(_PE)
    idx = (x.reshape(_BATCH, _NW, _NCHUNK, _CHUNK)
           .transpose(1, 2, 0, 3)
           .reshape(_NW, _NSTEP, _CHUNK))
    out = _embed_sc(table, idx, pe)
    return out.reshape(_BATCH, _TOKENS, _DIM)
